# Initial kernel scaffold; baseline (speedup 1.0000x reference)
import jax
import jax.numpy as jnp
from jax.experimental import pallas as pl


def kernel(Z, w):
    raise NotImplementedError("write your pallas kernel here")


# speedup vs baseline: 1.9885x; 1.9885x over previous
import functools

import jax
import jax.numpy as jnp
from jax import lax
from jax.experimental import pallas as pl
from jax.experimental.pallas import tpu as pltpu


def _fused_kernel(z_ref, wexp_ref, wsq_ref, o_ref, *, scale):
    z = z_ref[0].astype(jnp.float32)

    zw = jnp.dot(z, wexp_ref[...], preferred_element_type=jnp.float32)
    ss = jnp.dot(z * z, wsq_ref[...], preferred_element_type=jnp.float32)

    inv = lax.rsqrt(jnp.maximum(ss, 1e-24)) * scale
    f = (zw * inv).astype(jnp.bfloat16)

    gram = lax.dot_general(
        f, f,
        dimension_numbers=(((1,), (1,)), ((), ())),
        preferred_element_type=jnp.float32,
    )
    o_ref[0] = jnp.maximum(gram, 0.0).astype(o_ref.dtype)


def _round_up(x, m):
    return ((x + m - 1) // m) * m


def kernel(Z, w):
    B, N, D = Z.shape
    H, Dw = w.shape
    assert D == Dw, "w feature dim must match Z feature dim"
    HD = H * D

    n_pad = _round_up(N, 8) if N <= 128 else _round_up(N, 128)
    if n_pad != N:
        Z = jnp.pad(Z, ((0, 0), (0, n_pad - N), (0, 0)))

    w32 = w.astype(jnp.float32)
    eye = jnp.eye(D, dtype=jnp.float32)
    w_exp = (eye[None, :, :] * w32[:, None, :]).transpose(1, 0, 2).reshape(D, HD)
    w_sq = jnp.repeat((w32 * w32).T, D, axis=1)

    fused = functools.partial(_fused_kernel, scale=1.0 / (H ** 0.5))
    att = pl.pallas_call(
        fused,
        out_shape=jax.ShapeDtypeStruct((B, n_pad, n_pad), jnp.float32),
        grid=(B,),
        in_specs=[
            pl.BlockSpec((1, n_pad, D), lambda b: (b, 0, 0)),
            pl.BlockSpec((D, HD), lambda b: (0, 0)),
            pl.BlockSpec((D, HD), lambda b: (0, 0)),
        ],
        out_specs=pl.BlockSpec((1, n_pad, n_pad), lambda b: (b, 0, 0)),
        compiler_params=pltpu.CompilerParams(
            dimension_semantics=("parallel",),
            vmem_limit_bytes=48 * 1024 * 1024,
        ),
    )(Z, w_exp, w_sq)

    if n_pad != N:
        att = att[:, :N, :N]
    return att


# speedup vs baseline: 2.1815x; 1.0971x over previous
import functools

import jax
import jax.numpy as jnp
from jax import lax
from jax.experimental import pallas as pl
from jax.experimental.pallas import tpu as pltpu


def _fused_kernel(z_ref, wexp_ref, wsq_ref, o_ref, *, scale):
    for i in range(z_ref.shape[0]):
        z = z_ref[i].astype(jnp.float32)

        zw = jnp.dot(z, wexp_ref[...], preferred_element_type=jnp.float32)
        ss = jnp.dot(z * z, wsq_ref[...], preferred_element_type=jnp.float32)

        inv = lax.rsqrt(jnp.maximum(ss, 1e-24)) * scale
        f = (zw * inv).astype(jnp.bfloat16)

        gram = lax.dot_general(
            f, f,
            dimension_numbers=(((1,), (1,)), ((), ())),
            preferred_element_type=jnp.float32,
        )
        o_ref[i] = jnp.maximum(gram, 0.0).astype(o_ref.dtype)


def _round_up(x, m):
    return ((x + m - 1) // m) * m


def kernel(Z, w):
    B, N, D = Z.shape
    H, Dw = w.shape
    assert D == Dw, "w feature dim must match Z feature dim"
    HD = H * D

    n_pad = _round_up(N, 8) if N <= 128 else _round_up(N, 128)
    if n_pad != N:
        Z = jnp.pad(Z, ((0, 0), (0, n_pad - N), (0, 0)))

    w32 = w.astype(jnp.float32)
    eye = jnp.eye(D, dtype=jnp.float32)
    w_exp = (eye[None, :, :] * w32[:, None, :]).transpose(1, 0, 2).reshape(D, HD)
    w_sq = jnp.repeat((w32 * w32).T, D, axis=1)

    fused = functools.partial(_fused_kernel, scale=1.0 / (H ** 0.5))
    nb = 2 if B % 2 == 0 else 1
    att = pl.pallas_call(
        fused,
        out_shape=jax.ShapeDtypeStruct((B, n_pad, n_pad), jnp.float32),
        grid=(B // nb,),
        in_specs=[
            pl.BlockSpec((nb, n_pad, D), lambda b: (b, 0, 0)),
            pl.BlockSpec((D, HD), lambda b: (0, 0)),
            pl.BlockSpec((D, HD), lambda b: (0, 0)),
        ],
        out_specs=pl.BlockSpec((nb, n_pad, n_pad), lambda b: (b, 0, 0)),
        compiler_params=pltpu.CompilerParams(
            dimension_semantics=("parallel",),
            vmem_limit_bytes=48 * 1024 * 1024,
        ),
    )(Z, w_exp, w_sq)

    if n_pad != N:
        att = att[:, :N, :N]
    return att
